# Initial kernel scaffold; baseline (speedup 1.0000x reference)
import jax
import jax.numpy as jnp
from jax.experimental import pallas as pl


def kernel(inputs, tables):
    raise NotImplementedError("write your pallas kernel here")


# speedup vs baseline: 1.2090x; 1.2090x over previous
import functools

import jax
import jax.numpy as jnp
from jax import lax
from jax.experimental import pallas as pl
from jax.experimental.pallas import tpu as pltpu
from jax.experimental.pallas import tpu_sc as plsc

NUM_FIELDS = 26
VOCAB = 100000
EMBED_DIM = 32
BATCH = 16384

N_ROWS = BATCH * NUM_FIELDS
NC, NS = 2, 16
NW = NC * NS
ROWS_PER_W = N_ROWS // NW
K = 128
G = 8
CHUNKS_PER_W = ROWS_PER_W // K
GROUPS = CHUNKS_PER_W // G

_mesh = plsc.VectorSubcoreMesh(core_axis_name="c", subcore_axis_name="s")


@functools.partial(
    pl.kernel,
    out_type=jax.ShapeDtypeStruct((N_ROWS, EMBED_DIM), jnp.float32),
    mesh=_mesh,
    scratch_types=[
        pltpu.VMEM((CHUNKS_PER_W, K), jnp.int32),
        pltpu.VMEM((G * K, EMBED_DIM), jnp.float32),
        pltpu.SemaphoreType.DMA,
    ],
    compiler_params=pltpu.CompilerParams(use_tc_tiling_on_sc=False),
)
def _gather_kernel(table_hbm, idx_hbm, out_hbm, idx_v, rows_v, sem):
    wid = lax.axis_index("s") * NC + lax.axis_index("c")
    pltpu.sync_copy(idx_hbm.at[pl.ds(wid * CHUNKS_PER_W, CHUNKS_PER_W)], idx_v)
    base = wid * ROWS_PER_W

    def group(g, carry):
        copies = [
            pltpu.async_copy(
                table_hbm.at[idx_v.at[g * G + j]],
                rows_v.at[pl.ds(j * K, K)],
                sem,
            )
            for j in range(G)
        ]
        for c in copies:
            c.wait()
        pltpu.sync_copy(rows_v, out_hbm.at[pl.ds(base + g * (G * K), G * K)])
        return carry

    lax.fori_loop(0, GROUPS, group, 0)


def kernel(inputs, tables):
    offsets = (jnp.arange(NUM_FIELDS, dtype=jnp.int32) * VOCAB)[None, :]
    gidx = (inputs.astype(jnp.int32) + offsets).reshape(NW * CHUNKS_PER_W, K)
    flat_table = tables.reshape(NUM_FIELDS * VOCAB, EMBED_DIM)
    out = _gather_kernel(flat_table, gidx)
    return out.reshape(BATCH, NUM_FIELDS * EMBED_DIM)
